# baseline (speedup 1.0000x reference)
import functools

import jax
import jax.numpy as jnp
from jax import lax
from jax.experimental import pallas as pl
from jax.experimental.pallas import tpu as pltpu, tpu_sc as plsc

_D = 32
_B = 16384
_W = 128
_R = 8


@functools.cache
def _make_kernel(num_cores: int, num_subcores: int):
    nw = num_cores * num_subcores
    bpw = _B // nw
    nbatch = bpw // _R

    mesh = plsc.VectorSubcoreMesh(core_axis_name="c", subcore_axis_name="s")

    @functools.partial(
        pl.kernel,
        mesh=mesh,
        out_type=[
            jax.ShapeDtypeStruct((_D, _B), jnp.float32),
            jax.ShapeDtypeStruct((_D, _B), jnp.float32),
        ],
        scratch_types=[
            pltpu.VMEM((bpw,), jnp.int32),
            pltpu.VMEM((2 * _R, _D, _W), jnp.float32),
            pltpu.VMEM((_D, bpw), jnp.float32),
            pltpu.SemaphoreType.DMA,
            pltpu.SemaphoreType.DMA,
        ],
        compiler_params=pltpu.CompilerParams(needs_layout_passes=False),
    )
    def gather_kernel(uidx_hbm, iidx_hbm, utab_hbm, itab_hbm,
                      uout_hbm, iout_hbm,
                      idx_v, buf_v, o_v, sem0, sem1):
        wid = lax.axis_index("s") * num_cores + lax.axis_index("c")
        base = wid * bpw
        iota = lax.broadcasted_iota(jnp.int32, (16,), 0)
        zeros = jnp.zeros((16,), jnp.int32)
        sems = [sem0, sem1]

        def one_table(idx_hbm, tab_hbm, out_hbm):
            pltpu.sync_copy(idx_hbm.at[pl.ds(base, bpw)], idx_v)

            def fire(set_id, b, u):
                off = pl.multiple_of((u >> 7) * _W, _W)
                for tr in range(_D // 8):
                    pltpu.async_copy(
                        tab_hbm.at[pl.ds(8 * tr, 8), pl.ds(off, _W)],
                        buf_v.at[set_id * _R + b, pl.ds(8 * tr, 8)],
                        sems[set_id])

            def drain(set_id, b):
                pltpu.make_async_copy(
                    tab_hbm.at[:, pl.ds(0, _W)],
                    buf_v.at[set_id * _R + b], sems[set_id]).wait()

            def extract(set_id, b, u, i):
                lane = zeros + (u & (_W - 1))
                slot = zeros + (set_id * _R + b)
                col = zeros + i
                for h in range(_D // 16):
                    feats = iota + h * 16
                    x = plsc.load_gather(buf_v, [slot, feats, lane])
                    plsc.store_scatter(o_v, [feats, col], x)

            vec0 = idx_v[pl.ds(0, 16)]
            for b in range(_R):
                fire(0, b, vec0[b])

            def step(g, p):
                @pl.when(g + 1 < nbatch)
                def _():
                    nvec = idx_v[pl.ds(((g + 1) // 2) * 16, 16)]
                    for b in range(_R):
                        fire(1 - p, b, nvec[(1 - p) * _R + b])
                for b in range(_R):
                    drain(p, b)
                cvec = idx_v[pl.ds((g // 2) * 16, 16)]
                for b in range(_R):
                    extract(p, b, cvec[p * _R + b], g * _R + b)

            def outer(g, carry):
                @pl.when(lax.rem(g, 2) == 0)
                def _():
                    step(g, 0)

                @pl.when(lax.rem(g, 2) == 1)
                def _():
                    step(g, 1)

                return carry

            lax.fori_loop(0, nbatch, outer, 0)

            for f in range(_D):
                pltpu.sync_copy(o_v.at[f], out_hbm.at[f, pl.ds(base, bpw)])

        one_table(uidx_hbm, utab_hbm, uout_hbm)
        one_table(iidx_hbm, itab_hbm, iout_hbm)

    return gather_kernel


def kernel(user_indices, item_indices, embedding_user_weight, embedding_item_weight):
    info = plsc.get_sparse_core_info()
    k = _make_kernel(info.num_cores, info.num_subcores)
    uout_t, iout_t = k(
        user_indices.astype(jnp.int32),
        item_indices.astype(jnp.int32),
        embedding_user_weight.T,
        embedding_item_weight.T,
    )
    return (uout_t.T, iout_t.T)


# speedup vs baseline: 1.0176x; 1.0176x over previous
import functools

import jax
import jax.numpy as jnp
from jax import lax
from jax.experimental import pallas as pl
from jax.experimental.pallas import tpu as pltpu, tpu_sc as plsc

_D = 32
_B = 16384
_W = 128
_R = 8


@functools.cache
def _make_kernel(num_cores: int, num_subcores: int):
    nw = num_cores * num_subcores
    bpw = _B // nw
    nbatch = bpw // _R

    mesh = plsc.VectorSubcoreMesh(core_axis_name="c", subcore_axis_name="s")

    @functools.partial(
        pl.kernel,
        mesh=mesh,
        out_type=[
            jax.ShapeDtypeStruct((_D, _B), jnp.float32),
            jax.ShapeDtypeStruct((_D, _B), jnp.float32),
        ],
        scratch_types=[
            pltpu.VMEM((bpw,), jnp.int32),
            pltpu.VMEM((2 * _R, _D, _W), jnp.float32),
            pltpu.VMEM((2, _D, bpw), jnp.float32),
            pltpu.SemaphoreType.DMA,
            pltpu.SemaphoreType.DMA,
            pltpu.SemaphoreType.DMA,
        ],
        compiler_params=pltpu.CompilerParams(needs_layout_passes=False),
    )
    def gather_kernel(uidx_hbm, iidx_hbm, utab_hbm, itab_hbm,
                      uout_hbm, iout_hbm,
                      idx_v, buf_v, o_v, sem0, sem1, osem):
        wid = lax.axis_index("s") * num_cores + lax.axis_index("c")
        base = wid * bpw
        iota = lax.broadcasted_iota(jnp.int32, (16,), 0)
        zeros = jnp.zeros((16,), jnp.int32)
        sems = [sem0, sem1]

        def one_table(idx_hbm, tab_hbm, out_hbm, t):
            pltpu.sync_copy(idx_hbm.at[pl.ds(base, bpw)], idx_v)

            def fire(set_id, b, u):
                off = pl.multiple_of((u >> 7) * _W, _W)
                for tr in range(_D // 8):
                    pltpu.async_copy(
                        tab_hbm.at[pl.ds(8 * tr, 8), pl.ds(off, _W)],
                        buf_v.at[set_id * _R + b, pl.ds(8 * tr, 8)],
                        sems[set_id])

            def drain(set_id, b):
                pltpu.make_async_copy(
                    tab_hbm.at[:, pl.ds(0, _W)],
                    buf_v.at[set_id * _R + b], sems[set_id]).wait()

            def extract(set_id, b, u, i):
                lane = zeros + (u & (_W - 1))
                slot = zeros + (set_id * _R + b)
                col = zeros + i
                for h in range(_D // 16):
                    feats = iota + h * 16
                    x = plsc.load_gather(buf_v, [slot, feats, lane])
                    plsc.store_scatter(o_v.at[t], [feats, col], x)

            vec0 = idx_v[pl.ds(0, 16)]
            for b in range(_R):
                fire(0, b, vec0[b])

            def step(g, p):
                @pl.when(g + 1 < nbatch)
                def _():
                    nvec = idx_v[pl.ds(((g + 1) // 2) * 16, 16)]
                    for b in range(_R):
                        fire(1 - p, b, nvec[(1 - p) * _R + b])
                for b in range(_R):
                    drain(p, b)
                cvec = idx_v[pl.ds((g // 2) * 16, 16)]
                for b in range(_R):
                    extract(p, b, cvec[p * _R + b], g * _R + b)

            def outer(g, carry):
                @pl.when(lax.rem(g, 2) == 0)
                def _():
                    step(g, 0)

                @pl.when(lax.rem(g, 2) == 1)
                def _():
                    step(g, 1)

                return carry

            lax.fori_loop(0, nbatch, outer, 0)

            for f in range(_D):
                pltpu.async_copy(
                    o_v.at[t, f], out_hbm.at[f, pl.ds(base, bpw)], osem)

        def drain_out(out_hbm, t):
            for f in range(_D):
                pltpu.make_async_copy(
                    o_v.at[t, f], out_hbm.at[f, pl.ds(base, bpw)], osem
                ).wait()

        one_table(uidx_hbm, utab_hbm, uout_hbm, 0)
        one_table(iidx_hbm, itab_hbm, iout_hbm, 1)
        drain_out(uout_hbm, 0)
        drain_out(iout_hbm, 1)

    return gather_kernel


def kernel(user_indices, item_indices, embedding_user_weight, embedding_item_weight):
    info = plsc.get_sparse_core_info()
    k = _make_kernel(info.num_cores, info.num_subcores)
    uout_t, iout_t = k(
        user_indices.astype(jnp.int32),
        item_indices.astype(jnp.int32),
        embedding_user_weight.T,
        embedding_item_weight.T,
    )
    return (uout_t.T, iout_t.T)
